# baseline (speedup 1.0000x reference)
import jax
import jax.numpy as jnp
from jax import lax
from jax.experimental import pallas as pl
from jax.experimental.pallas import tpu as pltpu
from jax.experimental.pallas import tpu_sc as plsc

NUM_CORES = 2
NUM_SUBCORES = 16
NUM_WORKERS = NUM_CORES * NUM_SUBCORES
LANES = 16

B = 16384
C = 5
D = 64
W = 2 * D
CHUNK = 16
PAIRS = CHUNK * C
GROUPS = PAIRS // LANES
B_PER_W = B // NUM_WORKERS
NCHUNKS = B_PER_W // CHUNK


def _sc_body(trow_hbm, tcol_hbm, crow_hbm, ccol_hbm, tgt_table, ctx_table,
             out_hbm, trow_v, tcol_v, crow_v, ccol_v, tgt_rows, ctx_rows,
             out_v, sem_t, sem_c):
    wid = lax.axis_index("s") * NUM_CORES + lax.axis_index("c")
    lane_iota = lax.iota(jnp.int32, LANES)

    def chunk_body(ch, carry):
        base = wid * B_PER_W + ch * CHUNK
        pltpu.sync_copy(trow_hbm.at[pl.ds(base, CHUNK)], trow_v)
        pltpu.sync_copy(tcol_hbm.at[pl.ds(base, CHUNK)], tcol_v)
        pltpu.sync_copy(crow_hbm.at[pl.ds(base * C, PAIRS)], crow_v)
        pltpu.sync_copy(ccol_hbm.at[pl.ds(base * C, PAIRS)], ccol_v)
        cp_t = pltpu.async_copy(tgt_table.at[trow_v], tgt_rows, sem_t)
        cp_c = pltpu.async_copy(ctx_table.at[crow_v], ctx_rows, sem_c)
        cp_t.wait()
        cp_c.wait()

        for g in range(GROUPS):
            rvec = jnp.int32(g * LANES) + lane_iota
            ivec = lax.shift_right_logical(rvec * jnp.int32(52429), 18)
            crow_lane = jnp.int32(g * LANES) + lane_iota
            ccol = ccol_v[pl.ds(g * LANES, LANES)]
            tcol = plsc.load_gather(tcol_v, [ivec])
            acc = jnp.zeros((LANES,), jnp.float32)
            for d in range(D):
                cval = plsc.load_gather(ctx_rows, [crow_lane, ccol + d])
                tval = plsc.load_gather(tgt_rows, [ivec, tcol + d])
                acc = acc + cval * tval
            out_v[pl.ds(g * LANES, LANES)] = acc
        pltpu.sync_copy(out_v, out_hbm.at[pl.ds(base * C, PAIRS)])
        return carry

    lax.fori_loop(0, NCHUNKS, chunk_body, 0)


@jax.jit
def _sc_call(trow, tcol, crow, ccol, tgt_table, ctx_table):
    mesh = plsc.VectorSubcoreMesh(core_axis_name="c", subcore_axis_name="s")
    return pl.kernel(
        _sc_body,
        out_type=jax.ShapeDtypeStruct((B * C,), jnp.float32),
        mesh=mesh,
        compiler_params=pltpu.CompilerParams(needs_layout_passes=False),
        scratch_types=[
            pltpu.VMEM((CHUNK,), jnp.int32),
            pltpu.VMEM((CHUNK,), jnp.int32),
            pltpu.VMEM((PAIRS,), jnp.int32),
            pltpu.VMEM((PAIRS,), jnp.int32),
            pltpu.VMEM((CHUNK, W), jnp.float32),
            pltpu.VMEM((PAIRS, W), jnp.float32),
            pltpu.VMEM((PAIRS,), jnp.float32),
            pltpu.SemaphoreType.DMA,
            pltpu.SemaphoreType.DMA,
        ],
    )(trow, tcol, crow, ccol, tgt_table, ctx_table)


def kernel(target, context, target_table, context_table):
    tgt_idx = target.reshape(B)
    ctx_idx = context.reshape(B * C)
    trow = tgt_idx >> 1
    tcol = (tgt_idx & 1) * D
    crow = ctx_idx >> 1
    ccol = (ctx_idx & 1) * D
    vhalf = target_table.shape[0] // 2
    t2 = target_table.reshape(vhalf, W)
    c2 = context_table.reshape(vhalf, W)
    out = _sc_call(trow, tcol, crow, ccol, t2, c2)
    return out.reshape(B, C)
